# baseline (speedup 1.0000x reference)
import functools

import jax
import jax.numpy as jnp
from jax import lax
from jax.experimental import pallas as pl
from jax.experimental.pallas import tpu as pltpu
from jax.experimental.pallas import tpu_sc as plsc

B, C, H, W = 4, 384, 224, 224
D = H * W
R = B * C
NW = 32
RPW = R // NW


def _gather_body(x_hbm, idx_hbm, out_hbm, idx_v, buf_a, buf_b, sem_a, sem_b):
    wid = lax.axis_index("s") * 2 + lax.axis_index("c")
    base = wid * RPW
    pltpu.sync_copy(idx_hbm.at[pl.ds(base, RPW)], idx_v)

    def row_of(j):
        chunk = idx_v[pl.ds(16 * (j // 16), 16)]
        return chunk[j % 16]

    pltpu.make_async_copy(x_hbm.at[pl.ds(row_of(0), 1)], buf_a, sem_a).start()
    pltpu.make_async_copy(x_hbm.at[pl.ds(row_of(1), 1)], buf_b, sem_b).start()
    for g in range(RPW // 2):
        j = 2 * g
        pltpu.make_async_copy(x_hbm.at[pl.ds(row_of(j), 1)], buf_a, sem_a).wait()
        pltpu.sync_copy(buf_a, out_hbm.at[pl.ds(base + j, 1)])
        if j + 2 < RPW:
            pltpu.make_async_copy(
                x_hbm.at[pl.ds(row_of(j + 2), 1)], buf_a, sem_a).start()
        pltpu.make_async_copy(x_hbm.at[pl.ds(row_of(j + 1), 1)], buf_b, sem_b).wait()
        pltpu.sync_copy(buf_b, out_hbm.at[pl.ds(base + j + 1, 1)])
        if j + 3 < RPW:
            pltpu.make_async_copy(
                x_hbm.at[pl.ds(row_of(j + 3), 1)], buf_b, sem_b).start()


RCHUNK = 384
CAST_ROWS = 128
CAST_COLS = 3584


def _cast_body(x_ref, o_ref):
    o_ref[:] = x_ref[:].astype(jnp.bfloat16).T


def _tc_cast_bf16_t(x_flat):
    return pl.pallas_call(
        _cast_body,
        grid=(R // CAST_ROWS, D // CAST_COLS),
        in_specs=[pl.BlockSpec((CAST_ROWS, CAST_COLS), lambda i, j: (i, j))],
        out_specs=pl.BlockSpec((CAST_COLS, CAST_ROWS), lambda i, j: (j, i)),
        out_shape=jax.ShapeDtypeStruct((D, R), jnp.bfloat16),
    )(x_flat)


def _rank_body(p_ref, out_ref):
    b = pl.program_id(0)
    c = pl.program_id(1)
    p = p_ref[0, 0, :]
    pi = p[:, None]
    pj = p[None, :]
    ii = lax.broadcasted_iota(jnp.int32, (C, C), 0)
    jj = lax.broadcasted_iota(jnp.int32, (C, C), 1)
    beats = (pj > pi) | ((pj == pi) & (jj < ii))
    rank = jnp.sum(beats.astype(jnp.int32), axis=1)
    rr = lax.broadcasted_iota(jnp.int32, (RCHUNK, C), 0) + c * RCHUNK
    eq = rank[None, :] == rr
    jj2 = lax.broadcasted_iota(jnp.int32, (RCHUNK, C), 1)
    idx = jnp.sum(jnp.where(eq, jj2, 0), axis=1)
    out_ref[0, 0, :] = idx + b * C


def _tc_rank(p):
    out = pl.pallas_call(
        _rank_body,
        grid=(B, C // RCHUNK),
        in_specs=[pl.BlockSpec((1, 1, C), lambda b, c: (b, 0, 0))],
        out_specs=pl.BlockSpec((1, 1, RCHUNK), lambda b, c: (b, 0, c)),
        out_shape=jax.ShapeDtypeStruct((B, 1, C), jnp.int32),
    )(p.reshape(B, 1, C))
    return out.reshape(B, C)


def _sc_gather(x_flat, idx_flat):
    mesh = plsc.VectorSubcoreMesh(core_axis_name="c", subcore_axis_name="s")
    return pl.kernel(
        _gather_body,
        out_type=jax.ShapeDtypeStruct((R, D), jnp.float32),
        mesh=mesh,
        scratch_types=[
            pltpu.VMEM((RPW,), jnp.int32),
            pltpu.VMEM((1, D), jnp.float32),
            pltpu.VMEM((1, D), jnp.float32),
            pltpu.SemaphoreType.DMA,
            pltpu.SemaphoreType.DMA,
        ],
    )(x_flat, idx_flat)


def kernel(x, ratio, weight):
    x_flat = x.reshape(R, D)
    x_bf = jnp.transpose(_tc_cast_bf16_t(x_flat), (1, 0)).reshape(B * C, 1, H, W)
    out = jax.lax.conv_general_dilated(
        x_bf, weight.astype(jnp.bfloat16), (1, 1), 'VALID',
        dimension_numbers=('NCHW', 'OIHW', 'NCHW'),
        preferred_element_type=jnp.float32)
    out = jnp.abs(out)
    p = jnp.sum(jnp.sum(out, -1), -1).reshape(B, C)
    p = p * jnp.asarray(ratio, p.dtype)
    row_ids = _tc_rank(p).reshape(-1)
    sel = _sc_gather(x_flat, row_ids)
    return sel.reshape(B, C, H, W)


# speedup vs baseline: 1.0005x; 1.0005x over previous
import jax
import jax.numpy as jnp
from jax import lax
from jax.experimental import pallas as pl
from jax.experimental.pallas import tpu as pltpu
from jax.experimental.pallas import tpu_sc as plsc

B, C, H, W = 4, 384, 224, 224
D = H * W
R = B * C
NW = 32
RPW = R // NW


def _gather_body(x_hbm, idx_hbm, out_hbm, idx_v, buf_a, buf_b, sem_a, sem_b):
    wid = lax.axis_index("s") * 2 + lax.axis_index("c")
    base = wid * RPW
    pltpu.sync_copy(idx_hbm.at[pl.ds(base, RPW)], idx_v)

    def row_of(j):
        chunk = idx_v[pl.ds(16 * (j // 16), 16)]
        return chunk[j % 16]

    pltpu.make_async_copy(x_hbm.at[pl.ds(row_of(0), 1)], buf_a, sem_a).start()
    pltpu.make_async_copy(x_hbm.at[pl.ds(row_of(1), 1)], buf_b, sem_b).start()
    for g in range(RPW // 2):
        j = 2 * g
        pltpu.make_async_copy(x_hbm.at[pl.ds(row_of(j), 1)], buf_a, sem_a).wait()
        pltpu.sync_copy(buf_a, out_hbm.at[pl.ds(base + j, 1)])
        if j + 2 < RPW:
            pltpu.make_async_copy(
                x_hbm.at[pl.ds(row_of(j + 2), 1)], buf_a, sem_a).start()
        pltpu.make_async_copy(x_hbm.at[pl.ds(row_of(j + 1), 1)], buf_b, sem_b).wait()
        pltpu.sync_copy(buf_b, out_hbm.at[pl.ds(base + j + 1, 1)])
        if j + 3 < RPW:
            pltpu.make_async_copy(
                x_hbm.at[pl.ds(row_of(j + 3), 1)], buf_b, sem_b).start()


RCHUNK = 384
CAST_ROWS = 128
CAST_COLS = 3584


def _cast_body(x_ref, o_ref):
    o_ref[:] = x_ref[:].astype(jnp.bfloat16).T


def _tc_cast_bf16_t(x_flat):
    return pl.pallas_call(
        _cast_body,
        grid=(R // CAST_ROWS, D // CAST_COLS),
        in_specs=[pl.BlockSpec((CAST_ROWS, CAST_COLS), lambda i, j: (i, j))],
        out_specs=pl.BlockSpec((CAST_COLS, CAST_ROWS), lambda i, j: (j, i)),
        out_shape=jax.ShapeDtypeStruct((D, R), jnp.bfloat16),
    )(x_flat)


def _rank_body(p_ref, out_ref):
    b = pl.program_id(0)
    c = pl.program_id(1)
    p = p_ref[0, 0, :]
    pi = p[:, None]
    pj = p[None, :]
    ii = lax.broadcasted_iota(jnp.int32, (C, C), 0)
    jj = lax.broadcasted_iota(jnp.int32, (C, C), 1)
    beats = (pj > pi) | ((pj == pi) & (jj < ii))
    rank = jnp.sum(beats.astype(jnp.int32), axis=1)
    rr = lax.broadcasted_iota(jnp.int32, (RCHUNK, C), 0) + c * RCHUNK
    eq = rank[None, :] == rr
    jj2 = lax.broadcasted_iota(jnp.int32, (RCHUNK, C), 1)
    idx = jnp.sum(jnp.where(eq, jj2, 0), axis=1)
    out_ref[0, 0, :] = idx + b * C


def _tc_rank(p):
    out = pl.pallas_call(
        _rank_body,
        grid=(B, C // RCHUNK),
        in_specs=[pl.BlockSpec((1, 1, C), lambda b, c: (b, 0, 0))],
        out_specs=pl.BlockSpec((1, 1, RCHUNK), lambda b, c: (b, 0, c)),
        out_shape=jax.ShapeDtypeStruct((B, 1, C), jnp.int32),
    )(p.reshape(B, 1, C))
    return out.reshape(B, C)


def _sc_gather(x_flat, idx_flat):
    mesh = plsc.VectorSubcoreMesh(core_axis_name="c", subcore_axis_name="s")
    return pl.kernel(
        _gather_body,
        out_type=jax.ShapeDtypeStruct((R, D), jnp.float32),
        mesh=mesh,
        scratch_types=[
            pltpu.VMEM((RPW,), jnp.int32),
            pltpu.VMEM((1, D), jnp.float32),
            pltpu.VMEM((1, D), jnp.float32),
            pltpu.SemaphoreType.DMA,
            pltpu.SemaphoreType.DMA,
        ],
    )(x_flat, idx_flat)


def kernel(x, ratio, weight):
    x_flat = x.reshape(R, D)
    x_bf = jnp.transpose(_tc_cast_bf16_t(x_flat), (1, 0)).reshape(B * C, 1, H, W)
    out = jax.lax.conv_general_dilated(
        x_bf, weight.astype(jnp.bfloat16), (1, 1), 'VALID',
        dimension_numbers=('NCHW', 'OIHW', 'NCHW'),
        preferred_element_type=jnp.float32)
    out = jnp.abs(out)
    p = jnp.sum(jnp.sum(out, -1), -1).reshape(B, C)
    p = p * jnp.asarray(ratio, p.dtype)
    row_ids = _tc_rank(p).reshape(-1)
    sel = _sc_gather(x_flat, row_ids)
    return sel.reshape(B, C, H, W)
